# Initial kernel scaffold; baseline (speedup 1.0000x reference)
import jax
import jax.numpy as jnp
from jax.experimental import pallas as pl


def kernel(x, edge_index, W1, att_src1, att_dst1, b1, W2, att_src2, att_dst2, b2):
    raise NotImplementedError("write your pallas kernel here")


# speedup vs baseline: 28.3505x; 28.3505x over previous
import functools

import jax
import jax.numpy as jnp
from jax import lax
from jax.experimental import pallas as pl
from jax.experimental.pallas import tpu as pltpu
from jax.experimental.pallas import tpu_sc as plsc

_N = 10000
_E = 320000
_D = 128
_NR = 10240
_WG = 128
_WA = 80
_K = 128
_EC = 64
_NC = 2
_NS = 16
_NW = _NC * _NS
_EPRIME = _E + _N
_CH = -(-_EPRIME // (_NW * _EC))
_EPT = _CH * _EC
_EP = _NW * _EPT
_ROWS_PT = _NR // _NS
_BR = 512


def _dense1_body(x_ref, w_ref, g_ref):
    g_ref[...] = jnp.dot(x_ref[...], w_ref[...],
                         preferred_element_type=jnp.float32)


_dense1 = pl.pallas_call(
    _dense1_body,
    grid=(_NR // _BR,),
    in_specs=[pl.BlockSpec((_BR, _D), lambda i: (i, 0)),
              pl.BlockSpec((_D, _WG), lambda i: (0, 0))],
    out_specs=pl.BlockSpec((_BR, _WG), lambda i: (i, 0)),
    out_shape=jax.ShapeDtypeStruct((_NR, _WG), jnp.float32),
)


def _mid_body(a0_ref, a1_ref, r8_ref, b1_ref, w2_ref, g_ref):
    u = a0_ref[...] + a1_ref[...]
    num = u[:, :64]
    den = u[:, 64:72] + 1e-16
    dbc = jnp.dot(den, r8_ref[...], preferred_element_type=jnp.float32)
    h1 = num / dbc + b1_ref[0:1, :]
    z = jnp.where(h1 > 0.0, h1, jnp.exp(h1) - 1.0)
    g_ref[...] = jnp.dot(z, w2_ref[...], preferred_element_type=jnp.float32)


_mid = pl.pallas_call(
    _mid_body,
    grid=(_NR // _BR,),
    in_specs=[pl.BlockSpec((_BR, _WA), lambda i: (i, 0)),
              pl.BlockSpec((_BR, _WA), lambda i: (i, 0)),
              pl.BlockSpec((8, 64), lambda i: (0, 0)),
              pl.BlockSpec((8, 64), lambda i: (0, 0)),
              pl.BlockSpec((64, _WG), lambda i: (0, 0))],
    out_specs=pl.BlockSpec((_BR, _WG), lambda i: (i, 0)),
    out_shape=jax.ShapeDtypeStruct((_NR, _WG), jnp.float32),
)


def _fin_body(a0_ref, a1_ref, b2_ref, o_ref):
    u = a0_ref[...] + a1_ref[...]
    num = u[:, :64]
    den = u[:, 64:65] + 1e-16
    y = num / den + b2_ref[0:1, :]
    m = jnp.max(y, axis=1, keepdims=True)
    lse = m + jnp.log(jnp.sum(jnp.exp(y - m), axis=1, keepdims=True))
    o_ref[...] = y - lse


_fin = pl.pallas_call(
    _fin_body,
    grid=(_NR // _BR,),
    in_specs=[pl.BlockSpec((_BR, _WA), lambda i: (i, 0)),
              pl.BlockSpec((_BR, _WA), lambda i: (i, 0)),
              pl.BlockSpec((8, 64), lambda i: (0, 0))],
    out_specs=pl.BlockSpec((_BR, 64), lambda i: (i, 0)),
    out_shape=jax.ShapeDtypeStruct((_NR, 64), jnp.float32),
)


@functools.lru_cache(maxsize=1)
def _build_edge_pass():
    sc_mesh = plsc.VectorSubcoreMesh(
        core_axis_name="c", subcore_axis_name="s",
        num_cores=_NC, num_subcores=_NS)

    @functools.partial(
        pl.kernel,
        mesh=sc_mesh,
        out_type=jax.ShapeDtypeStruct((_NC, _NR, _WA), jnp.float32),
        scratch_types=[
            pltpu.VMEM((_K,), jnp.int32),
            pltpu.VMEM((_EC,), jnp.int32),
            pltpu.VMEM((_K, _WG), jnp.float32),
            pltpu.VMEM((_EC, _WA), jnp.float32),
            pltpu.VMEM_SHARED((_NR, _WA), jnp.float32),
            pltpu.SemaphoreType.DMA,
        ],
    )
    def edge_pass(g_hbm, ci_hbm, dst_hbm, zero_hbm, out_hbm,
                  ci_v, dst_v, g_v, o_v, acc, sem_g):
        c = lax.axis_index("c")
        s = lax.axis_index("s")
        wid = s * _NC + c

        def blk(j, carry):
            r = pl.ds(s * _ROWS_PT + j * _EC, _EC)
            pltpu.sync_copy(zero_hbm.at[r], o_v)
            pltpu.sync_copy(o_v, acc.at[r])
            return carry

        lax.fori_loop(0, _ROWS_PT // _EC, blk, 0)
        plsc.subcore_barrier()

        def chunk(i, carry):
            pltpu.sync_copy(ci_hbm.at[pl.ds((wid * _CH + i) * _K, _K)], ci_v)
            pltpu.sync_copy(dst_hbm.at[pl.ds(wid * _EPT + i * _EC, _EC)],
                            dst_v)
            cg = pltpu.async_copy(g_hbm.at[ci_v], g_v, sem_g)
            cg.wait()

            def edge(e, c2):
                a = g_v[e, pl.ds(64, 16)] + g_v[_EC + e, pl.ds(80, 16)]
                a = jnp.exp(jnp.maximum(a, 0.2 * a))
                o_v[e, pl.ds(64, 16)] = a
                o_v[e, pl.ds(0, 16)] = g_v[e, pl.ds(0, 16)] * a
                o_v[e, pl.ds(16, 16)] = g_v[e, pl.ds(16, 16)] * a
                o_v[e, pl.ds(32, 16)] = g_v[e, pl.ds(32, 16)] * a
                o_v[e, pl.ds(48, 16)] = g_v[e, pl.ds(48, 16)] * a
                return c2

            lax.fori_loop(0, _EC, edge, 0, unroll=2)
            pltpu.sync_copy(o_v, acc.at[dst_v], add=True)
            return carry

        lax.fori_loop(0, _CH, chunk, 0)
        plsc.subcore_barrier()

        def out_blk(j, carry):
            r = pl.ds(s * _ROWS_PT + j * _EC, _EC)
            pltpu.sync_copy(acc.at[r], o_v)
            pltpu.sync_copy(o_v, out_hbm.at[c, r])
            return carry

        lax.fori_loop(0, _ROWS_PT // _EC, out_blk, 0)

    return edge_pass


def _prep1(W1, att_src1, att_dst1):
    ip = jnp.arange(64)
    perm = (ip % 8) * 8 + ip // 8
    w1cm = W1[:, perm]
    a_src = (W1.reshape(_D, 8, 8) * att_src1[None, :, :]).sum(-1)
    a_dst = (W1.reshape(_D, 8, 8) * att_dst1[None, :, :]).sum(-1)
    dup = jnp.arange(16) % 8
    pad = jnp.zeros((_D, _WG - 96), jnp.float32)
    return jnp.concatenate([w1cm, a_src[:, dup], a_dst[:, dup], pad], axis=1)


def _prep2(W2, att_src2, att_dst2):
    ip = jnp.arange(64)
    perm = (ip % 8) * 8 + ip // 8
    w2cm = W2[perm, :]
    v_src = w2cm @ att_src2[0]
    v_dst = w2cm @ att_dst2[0]
    pad = jnp.zeros((64, _WG - 96), jnp.float32)
    return jnp.concatenate(
        [w2cm, jnp.tile(v_src[:, None], (1, 16)),
         jnp.tile(v_dst[:, None], (1, 16)), pad], axis=1)


def kernel(x, edge_index, W1, att_src1, att_dst1, b1, W2, att_src2, att_dst2, b2):
    ip = jnp.arange(64)
    perm = (ip % 8) * 8 + ip // 8

    xp = jnp.pad(x, ((0, _NR - _N), (0, 0)))
    g1 = _dense1(xp, _prep1(W1, att_src1, att_dst1))

    loop = jnp.arange(_N, dtype=jnp.int32)
    padi = jnp.full((_EP - _EPRIME,), _N, dtype=jnp.int32)
    src = jnp.concatenate([edge_index[0], loop, padi])
    dst = jnp.concatenate([edge_index[1], loop, padi])
    ci = jnp.concatenate([src.reshape(_NW * _CH, _EC),
                          dst.reshape(_NW * _CH, _EC)], axis=1).reshape(-1)
    zeros = jnp.zeros((_NR, _WA), jnp.float32)

    edge_pass = _build_edge_pass()
    acc1 = edge_pass(g1, ci, dst, zeros)

    r8 = (jnp.arange(8)[:, None] == (ip[None, :] % 8)).astype(jnp.float32)
    b1r = jnp.broadcast_to(b1[perm][None, :], (8, 64))
    g2 = _mid(acc1[0], acc1[1], r8, b1r, _prep2(W2, att_src2, att_dst2))

    acc2 = edge_pass(g2, ci, dst, zeros)

    b2r = jnp.broadcast_to(b2[None, :], (8, 64))
    out = _fin(acc2[0], acc2[1], b2r)
    return out[:_N]


# speedup vs baseline: 32.6622x; 1.1521x over previous
import functools

import jax
import jax.numpy as jnp
from jax import lax
from jax.experimental import pallas as pl
from jax.experimental.pallas import tpu as pltpu
from jax.experimental.pallas import tpu_sc as plsc

_N = 10000
_E = 320000
_D = 128
_NR = 10240
_WG = 128
_WA = 80
_K = 128
_EC = 64
_NC = 2
_NS = 16
_NW = _NC * _NS
_EPRIME = _E + _N
_CH = -(-_EPRIME // (_NW * _EC))
_EPT = _CH * _EC
_EP = _NW * _EPT
_ROWS_PT = _NR // _NS
_BR = 512


def _dense1_body(x_ref, w_ref, g_ref):
    g_ref[...] = jnp.dot(x_ref[...], w_ref[...],
                         preferred_element_type=jnp.float32)


_dense1 = pl.pallas_call(
    _dense1_body,
    grid=(_NR // _BR,),
    in_specs=[pl.BlockSpec((_BR, _D), lambda i: (i, 0)),
              pl.BlockSpec((_D, _WG), lambda i: (0, 0))],
    out_specs=pl.BlockSpec((_BR, _WG), lambda i: (i, 0)),
    out_shape=jax.ShapeDtypeStruct((_NR, _WG), jnp.float32),
)


def _mid_body(a0_ref, a1_ref, r8_ref, b1_ref, w2_ref, g_ref):
    u = a0_ref[...] + a1_ref[...]
    num = u[:, :64]
    den = u[:, 64:72] + 1e-16
    dbc = jnp.dot(den, r8_ref[...], preferred_element_type=jnp.float32)
    h1 = num / dbc + b1_ref[0:1, :]
    z = jnp.where(h1 > 0.0, h1, jnp.exp(h1) - 1.0)
    g_ref[...] = jnp.dot(z, w2_ref[...], preferred_element_type=jnp.float32)


_mid = pl.pallas_call(
    _mid_body,
    grid=(_NR // _BR,),
    in_specs=[pl.BlockSpec((_BR, _WA), lambda i: (i, 0)),
              pl.BlockSpec((_BR, _WA), lambda i: (i, 0)),
              pl.BlockSpec((8, 64), lambda i: (0, 0)),
              pl.BlockSpec((8, 64), lambda i: (0, 0)),
              pl.BlockSpec((64, _WG), lambda i: (0, 0))],
    out_specs=pl.BlockSpec((_BR, _WG), lambda i: (i, 0)),
    out_shape=jax.ShapeDtypeStruct((_NR, _WG), jnp.float32),
)


def _fin_body(a0_ref, a1_ref, b2_ref, o_ref):
    u = a0_ref[...] + a1_ref[...]
    num = u[:, :64]
    den = u[:, 64:65] + 1e-16
    y = num / den + b2_ref[0:1, :]
    m = jnp.max(y, axis=1, keepdims=True)
    lse = m + jnp.log(jnp.sum(jnp.exp(y - m), axis=1, keepdims=True))
    o_ref[...] = y - lse


_fin = pl.pallas_call(
    _fin_body,
    grid=(_NR // _BR,),
    in_specs=[pl.BlockSpec((_BR, _WA), lambda i: (i, 0)),
              pl.BlockSpec((_BR, _WA), lambda i: (i, 0)),
              pl.BlockSpec((8, 64), lambda i: (0, 0))],
    out_specs=pl.BlockSpec((_BR, 64), lambda i: (i, 0)),
    out_shape=jax.ShapeDtypeStruct((_NR, 64), jnp.float32),
)


@functools.lru_cache(maxsize=1)
def _build_edge_pass():
    sc_mesh = plsc.VectorSubcoreMesh(
        core_axis_name="c", subcore_axis_name="s",
        num_cores=_NC, num_subcores=_NS)

    @functools.partial(
        pl.kernel,
        mesh=sc_mesh,
        out_type=jax.ShapeDtypeStruct((_NC, _NR, _WA), jnp.float32),
        scratch_types=[
            pltpu.VMEM((_K,), jnp.int32),
            pltpu.VMEM((_K,), jnp.int32),
            pltpu.VMEM((_EC,), jnp.int32),
            pltpu.VMEM((_K, _WG), jnp.float32),
            pltpu.VMEM((_K, _WG), jnp.float32),
            pltpu.VMEM((_EC, _WA), jnp.float32),
            pltpu.VMEM_SHARED((_NR, _WA), jnp.float32),
            pltpu.SemaphoreType.DMA,
            pltpu.SemaphoreType.DMA,
        ],
    )
    def edge_pass(g_hbm, ci_hbm, dst_hbm, zero_hbm, out_hbm,
                  ci0, ci1, dst_v, gd0, gd1, o_v, acc, sem0, sem1):
        c = lax.axis_index("c")
        s = lax.axis_index("s")
        wid = s * _NC + c

        def blk(j, carry):
            r = pl.ds(s * _ROWS_PT + j * _EC, _EC)
            pltpu.sync_copy(zero_hbm.at[r], o_v)
            pltpu.sync_copy(o_v, acc.at[r])
            return carry

        lax.fori_loop(0, _ROWS_PT // _EC, blk, 0)
        plsc.subcore_barrier()

        cbase = wid * (_CH + 1)

        def consume(i, gd):
            pltpu.sync_copy(dst_hbm.at[pl.ds(wid * _EPT + i * _EC, _EC)],
                            dst_v)

            def edge(e, c2):
                a = gd[e, pl.ds(64, 16)] + gd[_EC + e, pl.ds(80, 16)]
                a = jnp.exp(jnp.maximum(a, 0.2 * a))
                o_v[e, pl.ds(64, 16)] = a
                o_v[e, pl.ds(0, 16)] = gd[e, pl.ds(0, 16)] * a
                o_v[e, pl.ds(16, 16)] = gd[e, pl.ds(16, 16)] * a
                o_v[e, pl.ds(32, 16)] = gd[e, pl.ds(32, 16)] * a
                o_v[e, pl.ds(48, 16)] = gd[e, pl.ds(48, 16)] * a
                return c2

            lax.fori_loop(0, _EC, edge, 0, unroll=2)
            pltpu.sync_copy(o_v, acc.at[dst_v], add=True)

        pltpu.sync_copy(ci_hbm.at[pl.ds(cbase * _K, _K)], ci0)
        pltpu.async_copy(g_hbm.at[ci0], gd0, sem0)

        def pair(ii, carry):
            i0 = 2 * ii
            pltpu.sync_copy(ci_hbm.at[pl.ds((cbase + i0 + 1) * _K, _K)], ci1)
            pltpu.async_copy(g_hbm.at[ci1], gd1, sem1)
            pltpu.make_async_copy(g_hbm.at[ci0], gd0, sem0).wait()
            consume(i0, gd0)
            pltpu.sync_copy(ci_hbm.at[pl.ds((cbase + i0 + 2) * _K, _K)], ci0)
            pltpu.async_copy(g_hbm.at[ci0], gd0, sem0)
            pltpu.make_async_copy(g_hbm.at[ci1], gd1, sem1).wait()
            consume(i0 + 1, gd1)
            return carry

        lax.fori_loop(0, _CH // 2, pair, 0)
        pltpu.make_async_copy(g_hbm.at[ci0], gd0, sem0).wait()
        plsc.subcore_barrier()

        def out_blk(j, carry):
            r = pl.ds(s * _ROWS_PT + j * _EC, _EC)
            pltpu.sync_copy(acc.at[r], o_v)
            pltpu.sync_copy(o_v, out_hbm.at[c, r])
            return carry

        lax.fori_loop(0, _ROWS_PT // _EC, out_blk, 0)

    return edge_pass


def _prep1(W1, att_src1, att_dst1):
    ip = jnp.arange(64)
    perm = (ip % 8) * 8 + ip // 8
    w1cm = W1[:, perm]
    a_src = (W1.reshape(_D, 8, 8) * att_src1[None, :, :]).sum(-1)
    a_dst = (W1.reshape(_D, 8, 8) * att_dst1[None, :, :]).sum(-1)
    dup = jnp.arange(16) % 8
    pad = jnp.zeros((_D, _WG - 96), jnp.float32)
    return jnp.concatenate([w1cm, a_src[:, dup], a_dst[:, dup], pad], axis=1)


def _prep2(W2, att_src2, att_dst2):
    ip = jnp.arange(64)
    perm = (ip % 8) * 8 + ip // 8
    w2cm = W2[perm, :]
    v_src = w2cm @ att_src2[0]
    v_dst = w2cm @ att_dst2[0]
    pad = jnp.zeros((64, _WG - 96), jnp.float32)
    return jnp.concatenate(
        [w2cm, jnp.tile(v_src[:, None], (1, 16)),
         jnp.tile(v_dst[:, None], (1, 16)), pad], axis=1)


def kernel(x, edge_index, W1, att_src1, att_dst1, b1, W2, att_src2, att_dst2, b2):
    ip = jnp.arange(64)
    perm = (ip % 8) * 8 + ip // 8

    xp = jnp.pad(x, ((0, _NR - _N), (0, 0)))
    g1 = _dense1(xp, _prep1(W1, att_src1, att_dst1))

    loop = jnp.arange(_N, dtype=jnp.int32)
    padi = jnp.full((_EP - _EPRIME,), _N, dtype=jnp.int32)
    src = jnp.concatenate([edge_index[0], loop, padi])
    dst = jnp.concatenate([edge_index[1], loop, padi])
    ci3 = jnp.concatenate([src.reshape(_NW, _CH, _EC),
                           dst.reshape(_NW, _CH, _EC)], axis=2)
    cpad = jnp.full((_NW, 1, _K), _N, dtype=jnp.int32)
    ci = jnp.concatenate([ci3, cpad], axis=1).reshape(-1)
    zeros = jnp.zeros((_NR, _WA), jnp.float32)

    edge_pass = _build_edge_pass()
    acc1 = edge_pass(g1, ci, dst, zeros)

    r8 = (jnp.arange(8)[:, None] == (ip[None, :] % 8)).astype(jnp.float32)
    b1r = jnp.broadcast_to(b1[perm][None, :], (8, 64))
    g2 = _mid(acc1[0], acc1[1], r8, b1r, _prep2(W2, att_src2, att_dst2))

    acc2 = edge_pass(g2, ci, dst, zeros)

    b2r = jnp.broadcast_to(b2[None, :], (8, 64))
    out = _fin(acc2[0], acc2[1], b2r)
    return out[:_N]
